# baseline (speedup 1.0000x reference)
import jax
import jax.numpy as jnp
from jax.experimental import pallas as pl
from jax.experimental.pallas import tpu as pltpu

_B = 128
_D = 512
_NHEAD = 8
_DH = _D // _NHEAD
_FF = 256
_OUT = (32, 32)


def _mm(a, b):
    return jax.lax.dot_general(a, b, (((1,), (0,)), ((), ())),
                               preferred_element_type=jnp.float32)


def _mm_t(a, b):
    return jax.lax.dot_general(a, b, (((1,), (1,)), ((), ())),
                               preferred_element_type=jnp.float32)


def _mm_ct(a, b):
    return jax.lax.dot_general(a, b, (((0,), (0,)), ((), ())),
                               preferred_element_type=jnp.float32)


def _sublane_tree_sum(e):
    acc = e[0:8, :]
    for i in range(1, 16):
        acc = acc + e[8 * i:8 * i + 8, :]
    acc = acc[0:4, :] + acc[4:8, :]
    acc = acc[0:2, :] + acc[2:4, :]
    return acc[0:1, :] + acc[1:2, :]


def _row_sum(x):
    c = x[:, 0:128] + x[:, 128:256]
    c = c + x[:, 256:384]
    c = c + x[:, 384:512]
    return _sublane_tree_sum(c.T).T


def _layernorm(x, g, b, eps=1e-5):
    m = _row_sum(x) * jnp.float32(1.0 / _D)
    c = x - m
    v = _row_sum(c * c) * jnp.float32(1.0 / _D)
    return c / jnp.sqrt(v + eps) * g + b


def _head_pair(qc, kc, vc, scale):
    outs = []
    for j in (slice(0, _DH), slice(_DH, 2 * _DH)):
        st = _mm_t(kc[:, j], qc[:, j]) * scale
        mxt = jnp.max(st, axis=0, keepdims=True)
        ssum = _sublane_tree_sum(jnp.exp(st - mxt))
        s = _mm_t(qc[:, j], kc[:, j]) * scale
        mx = jnp.max(s, axis=1, keepdims=True)
        w = jnp.exp(s - mx) / ssum.T
        outs.append(_mm(w, vc[:, j]))
    return jnp.concatenate(outs, axis=-1)


def _mha_heads(q, k, v, scale):
    outs = []
    for hp in range(_NHEAD // 2):
        c = slice(hp * 2 * _DH, (hp + 1) * 2 * _DH)
        outs.append(_head_pair(q[:, c], k[:, c], v[:, c], scale))
    return jnp.concatenate(outs, axis=-1)


def _vitt_kernel(x_ref,
                 sa_wq, sa_wk, sa_wv, sa_wo, sa_bq, sa_bk, sa_bv, sa_bo,
                 ca_wq, ca_wk, ca_wv, ca_wo, ca_bq, ca_bk, ca_bv, ca_bo,
                 w1, b1, w2, b2,
                 g1, be1, g2, be2, g3, be3,
                 wr1, br1, wr2, br2,
                 out_ref,
                 r_ref, x1_ref, qc_ref, outs_ref):
    x = x_ref[:]
    scale = jnp.float32(_DH ** -0.5)

    qs = _mm(x, sa_wq[:]) + sa_bq[:]
    ks = _mm(x, sa_wk[:]) + sa_bk[:]
    vs = _mm(x, sa_wv[:]) + sa_bv[:]
    sa_out = _mm(_mha_heads(qs, ks, vs, scale), sa_wo[:]) + sa_bo[:]
    x1 = _layernorm(x + sa_out, g1[:], be1[:])
    x1_ref[:] = x1
    qc_ref[:] = _mm(x1, ca_wq[:]) + ca_bq[:]
    r_ref[:] = jnp.zeros((_B, _D), jnp.float32)

    def step(t, carry):
        k = _mm_t(r_ref[:], ca_wk[:]) + ca_bk[:]
        v = _mm_t(r_ref[:], ca_wv[:]) + ca_bv[:]
        att = _mha_heads(qc_ref[:], k, v, scale)
        ca_out = _mm_t(att, ca_wo[:]) + ca_bo[:]
        x2 = _layernorm(x1_ref[:] + ca_out, g2[:], be2[:])
        ff = _mm_t(jnp.maximum(_mm_t(x2, w1[:]) + b1[:], 0.0), w2[:]) + b2[:]
        out = _layernorm(x2 + ff, g3[:], be3[:])
        r_new = r_ref[:] + out
        r_ref[:] = r_new
        outs_ref[pl.ds(t, 1), :, :] = r_new[0:1, :].reshape(1, 1, _D)
        return carry

    jax.lax.fori_loop(0, _B, step, 0, unroll=8)

    outs = outs_ref[:].reshape(_B, _D)
    h = _mm(outs, wr1[:]) + br1[:]
    out_ref[:] = _mm(h, wr2[:]) + br2[:]


def kernel(x, params):
    sa, ca = params["sa"], params["ca"]
    row = lambda a: a.reshape(1, -1)
    args = (
        x,
        sa["Wq"], sa["Wk"], sa["Wv"], sa["Wo"],
        row(sa["bq"]), row(sa["bk"]), row(sa["bv"]), row(sa["bo"]),
        ca["Wq"], ca["Wk"].T, ca["Wv"].T, ca["Wo"].T,
        row(ca["bq"]), row(ca["bk"]), row(ca["bv"]), row(ca["bo"]),
        params["W1"].T, row(params["b1"]), params["W2"].T, row(params["b2"]),
        row(params["g1"]), row(params["be1"]),
        row(params["g2"]), row(params["be2"]),
        row(params["g3"]), row(params["be3"]),
        params["Wr1"], row(params["br1"]), params["Wr2"], row(params["br2"]),
    )
    out = pl.pallas_call(
        _vitt_kernel,
        out_shape=jax.ShapeDtypeStruct((_B, _OUT[0] * _OUT[1]), jnp.float32),
        compiler_params=pltpu.CompilerParams(
            fuse_transposed_lhs_in_matmul=True,
        ),
        scratch_shapes=[
            pltpu.VMEM((_B, _D), jnp.float32),
            pltpu.VMEM((_B, _D), jnp.float32),
            pltpu.VMEM((_B, _D), jnp.float32),
            pltpu.VMEM((_B, 1, _D), jnp.float32),
        ],
    )(*args)
    return out.reshape(_B, *_OUT)


# speedup vs baseline: 1.0529x; 1.0529x over previous
import jax
import jax.numpy as jnp
from jax.experimental import pallas as pl
from jax.experimental.pallas import tpu as pltpu

_B = 128
_D = 512
_NHEAD = 8
_DH = _D // _NHEAD
_FF = 256
_OUT = (32, 32)


def _mm(a, b):
    return jax.lax.dot_general(a, b, (((1,), (0,)), ((), ())),
                               preferred_element_type=jnp.float32)


def _mm_t(a, b):
    return jax.lax.dot_general(a, b, (((1,), (1,)), ((), ())),
                               preferred_element_type=jnp.float32)


def _mm_ct(a, b):
    return jax.lax.dot_general(a, b, (((0,), (0,)), ((), ())),
                               preferred_element_type=jnp.float32)


def _sublane_tree_sum(e):
    acc = e[0:8, :]
    for i in range(1, 16):
        acc = acc + e[8 * i:8 * i + 8, :]
    acc = acc[0:4, :] + acc[4:8, :]
    acc = acc[0:2, :] + acc[2:4, :]
    return acc[0:1, :] + acc[1:2, :]


def _row_sum(x):
    c = x[:, 0:128] + x[:, 128:256]
    c = c + x[:, 256:384]
    c = c + x[:, 384:512]
    return _sublane_tree_sum(c.T).T


def _layernorm(x, g, b, eps=1e-5):
    m = _row_sum(x) * jnp.float32(1.0 / _D)
    c = x - m
    v = _row_sum(c * c) * jnp.float32(1.0 / _D)
    return c / jnp.sqrt(v + eps) * g + b


def _head_pair(qc, kc, vc, scale):
    outs = []
    for j in (slice(0, _DH), slice(_DH, 2 * _DH)):
        st = _mm_t(kc[:, j], qc[:, j]) * scale
        mxt = jnp.max(st, axis=0, keepdims=True)
        ssum = _sublane_tree_sum(jnp.exp(st - mxt))
        s = _mm_t(qc[:, j], kc[:, j]) * scale
        mx = jnp.max(s, axis=1, keepdims=True)
        w = jnp.exp(s - mx) / ssum.T
        outs.append(_mm(w, vc[:, j]))
    return jnp.concatenate(outs, axis=-1)


def _mha_heads(q, k, v, scale):
    outs = []
    for hp in range(_NHEAD // 2):
        c = slice(hp * 2 * _DH, (hp + 1) * 2 * _DH)
        outs.append(_head_pair(q[:, c], k[:, c], v[:, c], scale))
    return jnp.concatenate(outs, axis=-1)


def _vitt_kernel(x_ref,
                 sa_wq, sa_wk, sa_wv, sa_wo, sa_bq, sa_bk, sa_bv, sa_bo,
                 ca_wq, ca_wk, ca_wv, ca_wo, ca_bq, ca_bk, ca_bv, ca_bo,
                 w1, b1, w2, b2,
                 g1, be1, g2, be2, g3, be3,
                 wr1, br1, wr2, br2,
                 out_ref,
                 r_ref, x1_ref, qc_ref, outs_ref):
    x = x_ref[:]
    scale = jnp.float32(_DH ** -0.5)

    qs = _mm(x, sa_wq[:]) + sa_bq[:]
    ks = _mm(x, sa_wk[:]) + sa_bk[:]
    vs = _mm(x, sa_wv[:]) + sa_bv[:]
    sa_out = _mm(_mha_heads(qs, ks, vs, scale), sa_wo[:]) + sa_bo[:]
    x1 = _layernorm(x + sa_out, g1[:], be1[:])
    x1_ref[:] = x1
    qc_ref[:] = _mm(x1, ca_wq[:]) + ca_bq[:]
    r_ref[:] = jnp.zeros((_B, _D), jnp.float32)

    def step(t, carry):
        k = _mm(r_ref[:], ca_wk[:]) + ca_bk[:]
        v = _mm(r_ref[:], ca_wv[:]) + ca_bv[:]
        att = _mha_heads(qc_ref[:], k, v, scale)
        ca_out = _mm(att, ca_wo[:]) + ca_bo[:]
        x2 = _layernorm(x1_ref[:] + ca_out, g2[:], be2[:])
        ff = _mm(jnp.maximum(_mm(x2, w1[:]) + b1[:], 0.0), w2[:]) + b2[:]
        out = _layernorm(x2 + ff, g3[:], be3[:])
        r_new = r_ref[:] + out
        r_ref[:] = r_new
        outs_ref[pl.ds(t, 1), :, :] = r_new[0:1, :].reshape(1, 1, _D)
        return carry

    jax.lax.fori_loop(0, _B, step, 0, unroll=8)

    outs = outs_ref[:].reshape(_B, _D)
    h = _mm(outs, wr1[:]) + br1[:]
    out_ref[:] = _mm(h, wr2[:]) + br2[:]


def kernel(x, params):
    sa, ca = params["sa"], params["ca"]
    row = lambda a: a.reshape(1, -1)
    args = (
        x,
        sa["Wq"], sa["Wk"], sa["Wv"], sa["Wo"],
        row(sa["bq"]), row(sa["bk"]), row(sa["bv"]), row(sa["bo"]),
        ca["Wq"], ca["Wk"], ca["Wv"], ca["Wo"],
        row(ca["bq"]), row(ca["bk"]), row(ca["bv"]), row(ca["bo"]),
        params["W1"], row(params["b1"]), params["W2"], row(params["b2"]),
        row(params["g1"]), row(params["be1"]),
        row(params["g2"]), row(params["be2"]),
        row(params["g3"]), row(params["be3"]),
        params["Wr1"], row(params["br1"]), params["Wr2"], row(params["br2"]),
    )
    out = pl.pallas_call(
        _vitt_kernel,
        out_shape=jax.ShapeDtypeStruct((_B, _OUT[0] * _OUT[1]), jnp.float32),
        compiler_params=pltpu.CompilerParams(
            fuse_transposed_lhs_in_matmul=True,
        ),
        scratch_shapes=[
            pltpu.VMEM((_B, _D), jnp.float32),
            pltpu.VMEM((_B, _D), jnp.float32),
            pltpu.VMEM((_B, _D), jnp.float32),
            pltpu.VMEM((_B, 1, _D), jnp.float32),
        ],
    )(*args)
    return out.reshape(_B, *_OUT)
